# baseline (speedup 1.0000x reference)
import functools
import math

import jax
import jax.numpy as jnp
from jax import lax
from jax.experimental import pallas as pl
from jax.experimental.pallas import tpu as pltpu
from jax.experimental.pallas import tpu_sc as plsc

D = 64
SCALE = math.sqrt(D)
CHUNK = 128
NBUF = 4
LANES = 16


def _sc_kernel(num_chunks_per_worker, nc, ns):
  ngroups = num_chunks_per_worker // NBUF
  mesh = plsc.VectorSubcoreMesh(core_axis_name="c", subcore_axis_name="s")

  scratch = [pltpu.VMEM((num_chunks_per_worker, CHUNK), jnp.int32)]
  scratch += [pltpu.VMEM((CHUNK, D), jnp.float32) for _ in range(NBUF)]
  scratch += [pltpu.SemaphoreType.DMA for _ in range(2 * NBUF)]

  total_rows = num_chunks_per_worker * CHUNK * nc * ns

  @functools.partial(
      pl.kernel,
      out_type=jax.ShapeDtypeStruct((total_rows, D), jnp.float32),
      mesh=mesh,
      scratch_types=scratch,
      compiler_params=pltpu.CompilerParams(use_tc_tiling_on_sc=False),
  )
  def k(idx_hbm, table_hbm, out_hbm, idx_v, *rest):
    rows = rest[:NBUF]
    gsem = rest[NBUF:2 * NBUF]
    ssem = rest[2 * NBUF:]
    wid = lax.axis_index("s") * nc + lax.axis_index("c")
    base_chunk = wid * num_chunks_per_worker

    pltpu.sync_copy(idx_hbm.at[pl.ds(base_chunk, num_chunks_per_worker)],
                    idx_v)

    def gather_start(j, b):
      pltpu.async_copy(table_hbm.at[idx_v.at[j]], rows[b], gsem[b])

    def gather_wait(j, b):
      pltpu.make_async_copy(table_hbm.at[idx_v.at[j]], rows[b],
                            gsem[b]).wait()

    def scatter_start(j, b):
      dst = out_hbm.at[pl.ds((base_chunk + j) * CHUNK, CHUNK)]
      pltpu.async_copy(rows[b], dst, ssem[b])

    def scatter_wait(j, b):
      dst = out_hbm.at[pl.ds((base_chunk + j) * CHUNK, CHUNK)]
      pltpu.make_async_copy(rows[b], dst, ssem[b]).wait()

    def scale(b):
      buf = rows[b]

      @pl.loop(0, CHUNK, unroll=4)
      def _(r):
        for c in range(D // LANES):
          sl = (r, pl.ds(c * LANES, LANES))
          buf[sl] = buf[sl] * SCALE

    for b in range(NBUF):
      gather_start(b, b)

    @pl.loop(0, ngroups - 1)
    def _(g):
      for b in range(NBUF):
        j = g * NBUF + b
        gather_wait(j, b)
        scale(b)
        scatter_start(j, b)
        scatter_wait(j, b)
        gather_start(j + NBUF, b)

    for b in range(NBUF):
      j = (ngroups - 1) * NBUF + b
      gather_wait(j, b)
      scale(b)
      scatter_start(j, b)
      scatter_wait(j, b)

  return k


def kernel(x, table):
  xs, ts = x.shape, table.shape
  b_total = xs[0] * xs[1]
  info = plsc.get_sparse_core_info()
  nw = info.num_cores * info.num_subcores
  num_chunks_per_worker = b_total // (CHUNK * nw)
  idx = jnp.reshape(x.astype(jnp.int32), (b_total // CHUNK, CHUNK))
  k = _sc_kernel(num_chunks_per_worker, info.num_cores, info.num_subcores)
  out = k(idx, table)
  return jnp.reshape(out, (xs[0], xs[1], ts[1]))
